# baseline (speedup 1.0000x reference)
import functools

import jax
import jax.numpy as jnp
from jax import lax
from jax.experimental import pallas as pl
from jax.experimental.pallas import tpu as pltpu
from jax.experimental.pallas import tpu_sc as plsc

_NC = 2
_NS = 16
_NW = _NC * _NS
_CHUNK = 40
_NSLICE = 2


def _linear_body(x_ref, w_ref, b_ref, gh_ref, xa_ref, xb_ref):
    h = x_ref.shape[1]
    out = (
        jnp.dot(x_ref[...], w_ref[...], preferred_element_type=jnp.float32)
        + b_ref[...]
    )
    gh_ref[...] = out[:, :3 * h]
    xa_ref[...] = out[:, 3 * h:4 * h]
    xb_ref[...] = out[:, 4 * h:]


def _node_linear(x, wcat, bcat, block_rows, interpret=False):
    n, h = x.shape
    ho = wcat.shape[1]
    row = lambda w: pl.BlockSpec((block_rows, w), lambda i: (i, 0))
    return pl.pallas_call(
        _linear_body,
        grid=(n // block_rows,),
        in_specs=[
            row(h),
            pl.BlockSpec((h, ho), lambda i: (0, 0)),
            pl.BlockSpec((1, ho), lambda i: (0, 0)),
        ],
        out_specs=[row(3 * h), row(h), row(h)],
        out_shape=[
            jax.ShapeDtypeStruct((n, 3 * h), jnp.float32),
            jax.ShapeDtypeStruct((n, h), jnp.float32),
            jax.ShapeDtypeStruct((n, h), jnp.float32),
        ],
        interpret=interpret,
    )(x, wcat, bcat)


def _edge_mlp_body(g_ref, ea_ref, wc_ref, b1_ref, w2_ref, b2_ref, o_ref):
    pre = (
        g_ref[...]
        + jnp.dot(ea_ref[...], wc_ref[...], preferred_element_type=jnp.float32)
        + b1_ref[...]
    )
    hdn = jnp.maximum(pre, 0.0)
    o_ref[...] = (
        jnp.dot(hdn, w2_ref[...], preferred_element_type=jnp.float32) + b2_ref[...]
    )


def _edge_mlp(g, ea, wc, b1, w2, b2, block_rows, interpret=False):
    e, h = ea.shape
    row_spec = pl.BlockSpec((block_rows, h), lambda i: (i, 0))
    full = lambda shape: pl.BlockSpec(shape, lambda i: (0, 0))
    return pl.pallas_call(
        _edge_mlp_body,
        grid=(e // block_rows,),
        in_specs=[
            row_spec, row_spec,
            full((h, h)), full((1, h)), full((h, h)), full((1, h)),
        ],
        out_specs=row_spec,
        out_shape=jax.ShapeDtypeStruct((e, h), jnp.float32),
        interpret=interpret,
    )(g, ea, wc, b1, w2, b2)


def _gru(parts, gh, x, wih, bih, block_rows, interpret=False):
    n, h = x.shape
    nb = n // block_rows
    npart = len(parts)

    def body(*refs):
        part_refs = refs[:2 * npart]
        gh_ref, x_ref, wih_ref, bih_ref, o_ref = refs[2 * npart:]
        agg = part_refs[0][...]
        for p in part_refs[1:]:
            agg = agg + p[...]
        gi = (
            jnp.dot(agg, wih_ref[...], preferred_element_type=jnp.float32)
            + bih_ref[...]
        )
        ghv = gh_ref[...]
        r = jax.nn.sigmoid(gi[:, :h] + ghv[:, :h])
        z = jax.nn.sigmoid(gi[:, h:2 * h] + ghv[:, h:2 * h])
        nn = jnp.tanh(gi[:, 2 * h:] + r * ghv[:, 2 * h:])
        o_ref[...] = (1.0 - z) * nn + z * x_ref[...]

    row_spec = pl.BlockSpec((block_rows, h), lambda i: (i, 0))
    row3_spec = pl.BlockSpec((block_rows, 3 * h), lambda i: (i, 0))
    lo_spec = pl.BlockSpec((block_rows, h), lambda i: (i + nb, 0))
    full = lambda shape: pl.BlockSpec(shape, lambda i: (0, 0))
    part_specs, part_args = [], []
    for p in parts:
        part_specs += [row_spec, lo_spec]
        part_args += [p, p]
    return pl.pallas_call(
        body,
        grid=(nb,),
        in_specs=part_specs + [
            row3_spec, row_spec, full((h, 3 * h)), full((1, 3 * h)),
        ],
        out_specs=row_spec,
        out_shape=jax.ShapeDtypeStruct((n, h), jnp.float32),
        interpret=interpret,
    )(*part_args, gh, x, wih, bih)


def _make_gather(n, e, h):
    epw = e // _NW
    nch = epw // _CHUNK
    assert epw % _CHUNK == 0 and nch % 2 == 1
    mesh = plsc.VectorSubcoreMesh(core_axis_name="c", subcore_axis_name="s")

    @functools.partial(
        pl.kernel,
        out_type=jax.ShapeDtypeStruct((e, h), jnp.float32),
        mesh=mesh,
        scratch_types=[
            pltpu.VMEM((epw,), jnp.int32),
            pltpu.VMEM((epw,), jnp.int32),
            pltpu.VMEM((2, _CHUNK, h), jnp.float32),
            pltpu.VMEM((2, _CHUNK, h), jnp.float32),
            pltpu.SemaphoreType.DMA,
            pltpu.SemaphoreType.DMA,
            pltpu.SemaphoreType.DMA,
            pltpu.SemaphoreType.DMA,
        ],
    )
    def gather(xa_hbm, xb_hbm, src_hbm, dst_hbm, g_hbm,
               sidx, didx, srows, drows, gsem0, gsem1, wsem0, wsem1):
        wid = lax.axis_index("s") * _NC + lax.axis_index("c")
        base0 = wid * epw
        pltpu.sync_copy(src_hbm.at[pl.ds(base0, epw)], sidx)
        pltpu.sync_copy(dst_hbm.at[pl.ds(base0, epw)], didx)
        gsems = (gsem0, gsem1)
        wsems = (wsem0, wsem1)

        def start_gather(j, b):
            off = j * _CHUNK
            pltpu.async_copy(
                xa_hbm.at[sidx.at[pl.ds(off, _CHUNK)]], srows.at[b], gsems[b])
            pltpu.async_copy(
                xb_hbm.at[didx.at[pl.ds(off, _CHUNK)]], drows.at[b], gsems[b])

        def wait_gather(b):
            pltpu.make_async_copy(
                xa_hbm.at[sidx.at[pl.ds(0, _CHUNK)]], srows.at[b],
                gsems[b]).wait()
            pltpu.make_async_copy(
                xa_hbm.at[sidx.at[pl.ds(0, _CHUNK)]], drows.at[b],
                gsems[b]).wait()

        def wait_wb(b):
            pltpu.make_async_copy(
                srows.at[b], g_hbm.at[pl.ds(base0, _CHUNK)], wsems[b]).wait()

        def body(j, b, first=False):
            @pl.when(j + 1 < nch)
            def _prefetch():
                if not first:
                    wait_wb(1 - b)
                start_gather(j + 1, 1 - b)
            wait_gather(b)

            @pl.loop(0, _CHUNK)
            def _row(r):
                for c in range(h // 16):
                    sl = pl.ds(c * 16, 16)
                    srows[b, r, sl] += drows[b, r, sl]

            pltpu.async_copy(
                srows.at[b], g_hbm.at[pl.ds(base0 + j * _CHUNK, _CHUNK)],
                wsems[b])

        start_gather(0, 0)
        body(0, 0, first=True)

        @pl.loop(1, nch - 1, step=2)
        def _pair(j):
            body(j, 1)
            body(j + 1, 0)

        wait_wb(0)
        wait_wb(1)

    return gather


def _make_scatter(n, e, h):
    epw = e // _NW
    nch = epw // _CHUNK
    assert epw % _CHUNK == 0 and nch % 2 == 1
    rpt = (n // _NS) & ~7
    tail = n - _NS * rpt
    mesh = plsc.VectorSubcoreMesh(core_axis_name="c", subcore_axis_name="s")

    @functools.partial(
        pl.kernel,
        out_type=jax.ShapeDtypeStruct((_NC * n, h), jnp.float32),
        mesh=mesh,
        scratch_types=[
            pltpu.VMEM((nch, _CHUNK), jnp.int32),
            pltpu.VMEM((2, _CHUNK, h), jnp.float32),
            pltpu.VMEM_SHARED((n, h), jnp.float32),
            pltpu.SemaphoreType.DMA,
            pltpu.SemaphoreType.DMA,
        ],
    )
    def scatter(msg_hbm, dst3d_hbm, zero_hbm, out_hbm, idx, rows, acc,
                lsem0, lsem1):
        cid = lax.axis_index("c")
        sid = lax.axis_index("s")
        wid = sid * _NC + cid
        base0 = wid * epw
        lsems = (lsem0, lsem1)
        pltpu.sync_copy(dst3d_hbm.at[wid], idx)
        pltpu.sync_copy(
            zero_hbm.at[pl.ds(sid * rpt, rpt)],
            acc.at[pl.ds(sid * rpt, rpt)],
        )
        if tail:
            @pl.when(sid == 0)
            def _zero_tail():
                pltpu.sync_copy(
                    zero_hbm.at[pl.ds(_NS * rpt, tail)],
                    acc.at[pl.ds(_NS * rpt, tail)],
                )
        plsc.subcore_barrier()

        def start_load(j, b):
            pltpu.async_copy(
                msg_hbm.at[pl.ds(base0 + j * _CHUNK, _CHUNK)], rows.at[b],
                lsems[b])

        def wait_load(b):
            pltpu.make_async_copy(
                msg_hbm.at[pl.ds(base0, _CHUNK)], rows.at[b], lsems[b]).wait()

        def body(j, b):
            @pl.when(j + 1 < nch)
            def _prefetch():
                start_load(j + 1, 1 - b)
            wait_load(b)
            pltpu.sync_copy(rows.at[b], acc.at[idx.at[j]], add=True)

        start_load(0, 0)
        body(0, 0)

        @pl.loop(1, nch - 1, step=2)
        def _pair(j):
            body(j, 1)
            body(j + 1, 0)

        plsc.subcore_barrier()
        pltpu.sync_copy(
            acc.at[pl.ds(sid * rpt, rpt)],
            out_hbm.at[pl.ds(cid * n + sid * rpt, rpt)],
        )
        if tail:
            @pl.when(sid == 0)
            def _out_tail():
                pltpu.sync_copy(
                    acc.at[pl.ds(_NS * rpt, tail)],
                    out_hbm.at[pl.ds(cid * n + _NS * rpt, tail)],
                )

    return scatter


def kernel(x, edge_index, edge_attr, W1, b1, W2, b2, Wih, Whh, bih, bhh):
    n, h = x.shape
    e = edge_index.shape[1]
    src = edge_index[0].astype(jnp.int32)
    dst = edge_index[1].astype(jnp.int32)

    wa = W1[:, :h].T
    wb = W1[:, h:2 * h].T
    wc = W1[:, 2 * h:].T
    w2t = W2.T
    whht = Whh.T
    wiht = Wih.T
    wcat = jnp.concatenate([whht, wa, wb], axis=1)
    bcat = jnp.concatenate(
        [bhh, jnp.zeros((2 * h,), jnp.float32)]
    ).reshape(1, 5 * h)

    gh, xa, xb = _node_linear(x, wcat, bcat, block_rows=2000)

    es = e // _NSLICE
    gather_k = _make_gather(n, es, h)
    scatter_k = _make_scatter(n, es, h)
    zeros = jnp.zeros((n, h), jnp.float32)
    parts = []
    for i in range(_NSLICE):
        sl = slice(i * es, (i + 1) * es)
        g_i = gather_k(xa, xb, src[sl], dst[sl])
        msg_i = _edge_mlp(
            g_i, edge_attr[sl], wc, b1.reshape(1, h), w2t, b2.reshape(1, h),
            block_rows=2000,
        )
        dst3d_i = dst[sl].reshape(_NW, es // (_NW * _CHUNK), _CHUNK)
        parts.append(scatter_k(msg_i, dst3d_i, zeros))

    return _gru(parts, gh, x, wiht, bih.reshape(1, 3 * h), block_rows=2000)


# speedup vs baseline: 1.0432x; 1.0432x over previous
import functools

import jax
import jax.numpy as jnp
from jax import lax
from jax.experimental import pallas as pl
from jax.experimental.pallas import tpu as pltpu
from jax.experimental.pallas import tpu_sc as plsc

_NC = 2
_NS = 16
_NW = _NC * _NS
_CHUNK = 80
_NSLICE = 2


def _linear_body(x_ref, w_ref, b_ref, gh_ref, xa_ref, xb_ref):
    h = x_ref.shape[1]
    out = (
        jnp.dot(x_ref[...], w_ref[...], preferred_element_type=jnp.float32)
        + b_ref[...]
    )
    gh_ref[...] = out[:, :3 * h]
    xa_ref[...] = out[:, 3 * h:4 * h]
    xb_ref[...] = out[:, 4 * h:]


def _node_linear(x, wcat, bcat, block_rows, interpret=False):
    n, h = x.shape
    ho = wcat.shape[1]
    row = lambda w: pl.BlockSpec((block_rows, w), lambda i: (i, 0))
    return pl.pallas_call(
        _linear_body,
        grid=(n // block_rows,),
        in_specs=[
            row(h),
            pl.BlockSpec((h, ho), lambda i: (0, 0)),
            pl.BlockSpec((1, ho), lambda i: (0, 0)),
        ],
        out_specs=[row(3 * h), row(h), row(h)],
        out_shape=[
            jax.ShapeDtypeStruct((n, 3 * h), jnp.float32),
            jax.ShapeDtypeStruct((n, h), jnp.float32),
            jax.ShapeDtypeStruct((n, h), jnp.float32),
        ],
        interpret=interpret,
    )(x, wcat, bcat)


def _edge_mlp_body(g_ref, ea_ref, wc_ref, b1_ref, w2_ref, b2_ref, o_ref):
    pre = (
        g_ref[...]
        + jnp.dot(ea_ref[...], wc_ref[...], preferred_element_type=jnp.float32)
        + b1_ref[...]
    )
    hdn = jnp.maximum(pre, 0.0)
    o_ref[...] = (
        jnp.dot(hdn, w2_ref[...], preferred_element_type=jnp.float32) + b2_ref[...]
    )


def _edge_mlp(g, ea, wc, b1, w2, b2, block_rows, interpret=False):
    e, h = ea.shape
    row_spec = pl.BlockSpec((block_rows, h), lambda i: (i, 0))
    full = lambda shape: pl.BlockSpec(shape, lambda i: (0, 0))
    return pl.pallas_call(
        _edge_mlp_body,
        grid=(e // block_rows,),
        in_specs=[
            row_spec, row_spec,
            full((h, h)), full((1, h)), full((h, h)), full((1, h)),
        ],
        out_specs=row_spec,
        out_shape=jax.ShapeDtypeStruct((e, h), jnp.float32),
        interpret=interpret,
    )(g, ea, wc, b1, w2, b2)


def _gru(parts, gh, x, wih, bih, block_rows, interpret=False):
    n, h = x.shape
    nb = n // block_rows
    npart = len(parts)

    def body(*refs):
        part_refs = refs[:2 * npart]
        gh_ref, x_ref, wih_ref, bih_ref, o_ref = refs[2 * npart:]
        agg = part_refs[0][...]
        for p in part_refs[1:]:
            agg = agg + p[...]
        gi = (
            jnp.dot(agg, wih_ref[...], preferred_element_type=jnp.float32)
            + bih_ref[...]
        )
        ghv = gh_ref[...]
        r = jax.nn.sigmoid(gi[:, :h] + ghv[:, :h])
        z = jax.nn.sigmoid(gi[:, h:2 * h] + ghv[:, h:2 * h])
        nn = jnp.tanh(gi[:, 2 * h:] + r * ghv[:, 2 * h:])
        o_ref[...] = (1.0 - z) * nn + z * x_ref[...]

    row_spec = pl.BlockSpec((block_rows, h), lambda i: (i, 0))
    row3_spec = pl.BlockSpec((block_rows, 3 * h), lambda i: (i, 0))
    lo_spec = pl.BlockSpec((block_rows, h), lambda i: (i + nb, 0))
    full = lambda shape: pl.BlockSpec(shape, lambda i: (0, 0))
    part_specs, part_args = [], []
    for p in parts:
        part_specs += [row_spec, lo_spec]
        part_args += [p, p]
    return pl.pallas_call(
        body,
        grid=(nb,),
        in_specs=part_specs + [
            row3_spec, row_spec, full((h, 3 * h)), full((1, 3 * h)),
        ],
        out_specs=row_spec,
        out_shape=jax.ShapeDtypeStruct((n, h), jnp.float32),
        interpret=interpret,
    )(*part_args, gh, x, wih, bih)


def _make_gather(n, e, h):
    epw = e // _NW
    nch = epw // _CHUNK
    assert epw % _CHUNK == 0
    mesh = plsc.VectorSubcoreMesh(core_axis_name="c", subcore_axis_name="s")

    @functools.partial(
        pl.kernel,
        out_type=jax.ShapeDtypeStruct((e, h), jnp.float32),
        mesh=mesh,
        scratch_types=[
            pltpu.VMEM((epw,), jnp.int32),
            pltpu.VMEM((epw,), jnp.int32),
            pltpu.VMEM((2, _CHUNK, h), jnp.float32),
            pltpu.VMEM((2, _CHUNK, h), jnp.float32),
            pltpu.SemaphoreType.DMA,
            pltpu.SemaphoreType.DMA,
            pltpu.SemaphoreType.DMA,
            pltpu.SemaphoreType.DMA,
        ],
    )
    def gather(xa_hbm, xb_hbm, src_hbm, dst_hbm, g_hbm,
               sidx, didx, srows, drows, gsem0, gsem1, wsem0, wsem1):
        wid = lax.axis_index("s") * _NC + lax.axis_index("c")
        base0 = wid * epw
        pltpu.sync_copy(src_hbm.at[pl.ds(base0, epw)], sidx)
        pltpu.sync_copy(dst_hbm.at[pl.ds(base0, epw)], didx)
        gsems = (gsem0, gsem1)
        wsems = (wsem0, wsem1)

        def start_gather(j, b):
            off = j * _CHUNK
            pltpu.async_copy(
                xa_hbm.at[sidx.at[pl.ds(off, _CHUNK)]], srows.at[b], gsems[b])
            pltpu.async_copy(
                xb_hbm.at[didx.at[pl.ds(off, _CHUNK)]], drows.at[b], gsems[b])

        def wait_gather(b):
            pltpu.make_async_copy(
                xa_hbm.at[sidx.at[pl.ds(0, _CHUNK)]], srows.at[b],
                gsems[b]).wait()
            pltpu.make_async_copy(
                xa_hbm.at[sidx.at[pl.ds(0, _CHUNK)]], drows.at[b],
                gsems[b]).wait()

        def wait_wb(b):
            pltpu.make_async_copy(
                srows.at[b], g_hbm.at[pl.ds(base0, _CHUNK)], wsems[b]).wait()

        def body(j, b, first=False):
            @pl.when(j + 1 < nch)
            def _prefetch():
                if not first:
                    wait_wb(1 - b)
                start_gather(j + 1, 1 - b)
            wait_gather(b)

            @pl.loop(0, _CHUNK, step=8)
            def _row(r):
                for u in range(8):
                    for c in range(h // 16):
                        sl = pl.ds(c * 16, 16)
                        srows[b, r + u, sl] += drows[b, r + u, sl]

            pltpu.async_copy(
                srows.at[b], g_hbm.at[pl.ds(base0 + j * _CHUNK, _CHUNK)],
                wsems[b])

        start_gather(0, 0)
        body(0, 0, first=True)

        @pl.loop(1, 2 * ((nch - 1) // 2), step=2)
        def _pair(j):
            body(j, 1)
            body(j + 1, 0)

        if nch % 2 == 0:
            body(nch - 1, 1)
        wait_wb(0)
        wait_wb(1)

    return gather


def _make_scatter(n, e, h):
    epw = e // _NW
    nch = epw // _CHUNK
    assert epw % _CHUNK == 0
    rpt = (n // _NS) & ~7
    tail = n - _NS * rpt
    mesh = plsc.VectorSubcoreMesh(core_axis_name="c", subcore_axis_name="s")

    @functools.partial(
        pl.kernel,
        out_type=jax.ShapeDtypeStruct((_NC * n, h), jnp.float32),
        mesh=mesh,
        scratch_types=[
            pltpu.VMEM((nch, _CHUNK), jnp.int32),
            pltpu.VMEM((2, _CHUNK, h), jnp.float32),
            pltpu.VMEM_SHARED((n, h), jnp.float32),
            pltpu.SemaphoreType.DMA,
            pltpu.SemaphoreType.DMA,
        ],
    )
    def scatter(msg_hbm, dst3d_hbm, zero_hbm, out_hbm, idx, rows, acc,
                lsem0, lsem1):
        cid = lax.axis_index("c")
        sid = lax.axis_index("s")
        wid = sid * _NC + cid
        base0 = wid * epw
        lsems = (lsem0, lsem1)
        pltpu.sync_copy(dst3d_hbm.at[wid], idx)
        pltpu.sync_copy(
            zero_hbm.at[pl.ds(sid * rpt, rpt)],
            acc.at[pl.ds(sid * rpt, rpt)],
        )
        if tail:
            @pl.when(sid == 0)
            def _zero_tail():
                pltpu.sync_copy(
                    zero_hbm.at[pl.ds(_NS * rpt, tail)],
                    acc.at[pl.ds(_NS * rpt, tail)],
                )
        plsc.subcore_barrier()

        def start_load(j, b):
            pltpu.async_copy(
                msg_hbm.at[pl.ds(base0 + j * _CHUNK, _CHUNK)], rows.at[b],
                lsems[b])

        def wait_load(b):
            pltpu.make_async_copy(
                msg_hbm.at[pl.ds(base0, _CHUNK)], rows.at[b], lsems[b]).wait()

        def body(j, b):
            @pl.when(j + 1 < nch)
            def _prefetch():
                start_load(j + 1, 1 - b)
            wait_load(b)
            pltpu.sync_copy(rows.at[b], acc.at[idx.at[j]], add=True)

        start_load(0, 0)
        body(0, 0)

        @pl.loop(1, 2 * ((nch - 1) // 2), step=2)
        def _pair(j):
            body(j, 1)
            body(j + 1, 0)

        if nch % 2 == 0:
            body(nch - 1, 1)
        plsc.subcore_barrier()
        pltpu.sync_copy(
            acc.at[pl.ds(sid * rpt, rpt)],
            out_hbm.at[pl.ds(cid * n + sid * rpt, rpt)],
        )
        if tail:
            @pl.when(sid == 0)
            def _out_tail():
                pltpu.sync_copy(
                    acc.at[pl.ds(_NS * rpt, tail)],
                    out_hbm.at[pl.ds(cid * n + _NS * rpt, tail)],
                )

    return scatter


def kernel(x, edge_index, edge_attr, W1, b1, W2, b2, Wih, Whh, bih, bhh):
    n, h = x.shape
    e = edge_index.shape[1]
    src = edge_index[0].astype(jnp.int32)
    dst = edge_index[1].astype(jnp.int32)

    wa = W1[:, :h].T
    wb = W1[:, h:2 * h].T
    wc = W1[:, 2 * h:].T
    w2t = W2.T
    whht = Whh.T
    wiht = Wih.T
    wcat = jnp.concatenate([whht, wa, wb], axis=1)
    bcat = jnp.concatenate(
        [bhh, jnp.zeros((2 * h,), jnp.float32)]
    ).reshape(1, 5 * h)

    gh, xa, xb = _node_linear(x, wcat, bcat, block_rows=2000)

    wchunk = _NW * _CHUNK
    tw = e // wchunk
    cuts = [tw * i // _NSLICE for i in range(_NSLICE + 1)]
    zeros = jnp.zeros((n, h), jnp.float32)
    parts = []
    for i in range(_NSLICE):
        lo, es = cuts[i] * wchunk, (cuts[i + 1] - cuts[i]) * wchunk
        sl = slice(lo, lo + es)
        g_i = _make_gather(n, es, h)(xa, xb, src[sl], dst[sl])
        msg_i = _edge_mlp(
            g_i, edge_attr[sl], wc, b1.reshape(1, h), w2t, b2.reshape(1, h),
            block_rows=wchunk,
        )
        dst3d_i = dst[sl].reshape(_NW, es // wchunk, _CHUNK)
        parts.append(_make_scatter(n, es, h)(msg_i, dst3d_i, zeros))

    return _gru(parts, gh, x, wiht, bih.reshape(1, 3 * h), block_rows=2000)


# speedup vs baseline: 1.1195x; 1.0732x over previous
import functools

import jax
import jax.numpy as jnp
from jax import lax
from jax.experimental import pallas as pl
from jax.experimental.pallas import tpu as pltpu
from jax.experimental.pallas import tpu_sc as plsc

_NC = 2
_NS = 16
_NW = _NC * _NS
_CHUNK = 80
_NSLICE = 1


def _linear_body(x_ref, w_ref, b_ref, gh_ref, xa_ref, xb_ref):
    h = x_ref.shape[1]
    out = (
        jnp.dot(x_ref[...], w_ref[...], preferred_element_type=jnp.float32)
        + b_ref[...]
    )
    gh_ref[...] = out[:, :3 * h]
    xa_ref[...] = out[:, 3 * h:4 * h]
    xb_ref[...] = out[:, 4 * h:]


def _node_linear(x, wcat, bcat, block_rows, interpret=False):
    n, h = x.shape
    ho = wcat.shape[1]
    row = lambda w: pl.BlockSpec((block_rows, w), lambda i: (i, 0))
    return pl.pallas_call(
        _linear_body,
        grid=(n // block_rows,),
        in_specs=[
            row(h),
            pl.BlockSpec((h, ho), lambda i: (0, 0)),
            pl.BlockSpec((1, ho), lambda i: (0, 0)),
        ],
        out_specs=[row(3 * h), row(h), row(h)],
        out_shape=[
            jax.ShapeDtypeStruct((n, 3 * h), jnp.float32),
            jax.ShapeDtypeStruct((n, h), jnp.float32),
            jax.ShapeDtypeStruct((n, h), jnp.float32),
        ],
        interpret=interpret,
    )(x, wcat, bcat)


def _edge_mlp_body(g_ref, ea_ref, wc_ref, b1_ref, w2_ref, b2_ref, o_ref):
    pre = (
        g_ref[...]
        + jnp.dot(ea_ref[...], wc_ref[...], preferred_element_type=jnp.float32)
        + b1_ref[...]
    )
    hdn = jnp.maximum(pre, 0.0)
    o_ref[...] = (
        jnp.dot(hdn, w2_ref[...], preferred_element_type=jnp.float32) + b2_ref[...]
    )


def _edge_mlp(g, ea, wc, b1, w2, b2, block_rows, interpret=False):
    e, h = ea.shape
    row_spec = pl.BlockSpec((block_rows, h), lambda i: (i, 0))
    full = lambda shape: pl.BlockSpec(shape, lambda i: (0, 0))
    return pl.pallas_call(
        _edge_mlp_body,
        grid=(e // block_rows,),
        in_specs=[
            row_spec, row_spec,
            full((h, h)), full((1, h)), full((h, h)), full((1, h)),
        ],
        out_specs=row_spec,
        out_shape=jax.ShapeDtypeStruct((e, h), jnp.float32),
        interpret=interpret,
    )(g, ea, wc, b1, w2, b2)


def _gru(parts, gh, x, wih, bih, block_rows, interpret=False):
    n, h = x.shape
    nb = n // block_rows
    npart = len(parts)

    def body(*refs):
        part_refs = refs[:2 * npart]
        gh_ref, x_ref, wih_ref, bih_ref, o_ref = refs[2 * npart:]
        agg = part_refs[0][...]
        for p in part_refs[1:]:
            agg = agg + p[...]
        gi = (
            jnp.dot(agg, wih_ref[...], preferred_element_type=jnp.float32)
            + bih_ref[...]
        )
        ghv = gh_ref[...]
        r = jax.nn.sigmoid(gi[:, :h] + ghv[:, :h])
        z = jax.nn.sigmoid(gi[:, h:2 * h] + ghv[:, h:2 * h])
        nn = jnp.tanh(gi[:, 2 * h:] + r * ghv[:, 2 * h:])
        o_ref[...] = (1.0 - z) * nn + z * x_ref[...]

    row_spec = pl.BlockSpec((block_rows, h), lambda i: (i, 0))
    row3_spec = pl.BlockSpec((block_rows, 3 * h), lambda i: (i, 0))
    lo_spec = pl.BlockSpec((block_rows, h), lambda i: (i + nb, 0))
    full = lambda shape: pl.BlockSpec(shape, lambda i: (0, 0))
    part_specs, part_args = [], []
    for p in parts:
        part_specs += [row_spec, lo_spec]
        part_args += [p, p]
    return pl.pallas_call(
        body,
        grid=(nb,),
        in_specs=part_specs + [
            row3_spec, row_spec, full((h, 3 * h)), full((1, 3 * h)),
        ],
        out_specs=row_spec,
        out_shape=jax.ShapeDtypeStruct((n, h), jnp.float32),
        interpret=interpret,
    )(*part_args, gh, x, wih, bih)


def _make_gather(n, e, h):
    epw = e // _NW
    nch = epw // _CHUNK
    assert epw % _CHUNK == 0
    mesh = plsc.VectorSubcoreMesh(core_axis_name="c", subcore_axis_name="s")

    @functools.partial(
        pl.kernel,
        out_type=jax.ShapeDtypeStruct((e, h), jnp.float32),
        mesh=mesh,
        scratch_types=[
            pltpu.VMEM((epw,), jnp.int32),
            pltpu.VMEM((epw,), jnp.int32),
            pltpu.VMEM((2, _CHUNK, h), jnp.float32),
            pltpu.VMEM((2, _CHUNK, h), jnp.float32),
            pltpu.SemaphoreType.DMA,
            pltpu.SemaphoreType.DMA,
            pltpu.SemaphoreType.DMA,
            pltpu.SemaphoreType.DMA,
        ],
    )
    def gather(xa_hbm, xb_hbm, src_hbm, dst_hbm, g_hbm,
               sidx, didx, srows, drows, gsem0, gsem1, wsem0, wsem1):
        wid = lax.axis_index("s") * _NC + lax.axis_index("c")
        base0 = wid * epw
        pltpu.sync_copy(src_hbm.at[pl.ds(base0, epw)], sidx)
        pltpu.sync_copy(dst_hbm.at[pl.ds(base0, epw)], didx)
        gsems = (gsem0, gsem1)
        wsems = (wsem0, wsem1)

        def start_gather(j, b):
            off = j * _CHUNK
            pltpu.async_copy(
                xa_hbm.at[sidx.at[pl.ds(off, _CHUNK)]], srows.at[b], gsems[b])
            pltpu.async_copy(
                xb_hbm.at[didx.at[pl.ds(off, _CHUNK)]], drows.at[b], gsems[b])

        def wait_gather(b):
            pltpu.make_async_copy(
                xa_hbm.at[sidx.at[pl.ds(0, _CHUNK)]], srows.at[b],
                gsems[b]).wait()
            pltpu.make_async_copy(
                xa_hbm.at[sidx.at[pl.ds(0, _CHUNK)]], drows.at[b],
                gsems[b]).wait()

        def wait_wb(b):
            pltpu.make_async_copy(
                srows.at[b], g_hbm.at[pl.ds(base0, _CHUNK)], wsems[b]).wait()

        def body(j, b, first=False):
            @pl.when(j + 1 < nch)
            def _prefetch():
                if not first:
                    wait_wb(1 - b)
                start_gather(j + 1, 1 - b)
            wait_gather(b)

            @pl.loop(0, _CHUNK, step=8)
            def _row(r):
                for u in range(8):
                    for c in range(h // 16):
                        sl = pl.ds(c * 16, 16)
                        srows[b, r + u, sl] += drows[b, r + u, sl]

            pltpu.async_copy(
                srows.at[b], g_hbm.at[pl.ds(base0 + j * _CHUNK, _CHUNK)],
                wsems[b])

        start_gather(0, 0)
        body(0, 0, first=True)

        @pl.loop(1, 2 * ((nch - 1) // 2), step=2)
        def _pair(j):
            body(j, 1)
            body(j + 1, 0)

        if nch % 2 == 0:
            body(nch - 1, 1)
        wait_wb(0)
        wait_wb(1)

    return gather


def _make_scatter(n, e, h):
    epw = e // _NW
    nch = epw // _CHUNK
    assert epw % _CHUNK == 0
    rpt = (n // _NS) & ~7
    tail = n - _NS * rpt
    mesh = plsc.VectorSubcoreMesh(core_axis_name="c", subcore_axis_name="s")

    @functools.partial(
        pl.kernel,
        out_type=jax.ShapeDtypeStruct((_NC * n, h), jnp.float32),
        mesh=mesh,
        scratch_types=[
            pltpu.VMEM((nch, _CHUNK), jnp.int32),
            pltpu.VMEM((2, _CHUNK, h), jnp.float32),
            pltpu.VMEM_SHARED((n, h), jnp.float32),
            pltpu.SemaphoreType.DMA,
            pltpu.SemaphoreType.DMA,
        ],
    )
    def scatter(msg_hbm, dst3d_hbm, zero_hbm, out_hbm, idx, rows, acc,
                lsem0, lsem1):
        cid = lax.axis_index("c")
        sid = lax.axis_index("s")
        wid = sid * _NC + cid
        base0 = wid * epw
        lsems = (lsem0, lsem1)
        pltpu.sync_copy(dst3d_hbm.at[wid], idx)
        pltpu.sync_copy(
            zero_hbm.at[pl.ds(sid * rpt, rpt)],
            acc.at[pl.ds(sid * rpt, rpt)],
        )
        if tail:
            @pl.when(sid == 0)
            def _zero_tail():
                pltpu.sync_copy(
                    zero_hbm.at[pl.ds(_NS * rpt, tail)],
                    acc.at[pl.ds(_NS * rpt, tail)],
                )
        plsc.subcore_barrier()

        def start_load(j, b):
            pltpu.async_copy(
                msg_hbm.at[pl.ds(base0 + j * _CHUNK, _CHUNK)], rows.at[b],
                lsems[b])

        def wait_load(b):
            pltpu.make_async_copy(
                msg_hbm.at[pl.ds(base0, _CHUNK)], rows.at[b], lsems[b]).wait()

        def body(j, b):
            @pl.when(j + 1 < nch)
            def _prefetch():
                start_load(j + 1, 1 - b)
            wait_load(b)
            pltpu.sync_copy(rows.at[b], acc.at[idx.at[j]], add=True)

        start_load(0, 0)
        body(0, 0)

        @pl.loop(1, 2 * ((nch - 1) // 2), step=2)
        def _pair(j):
            body(j, 1)
            body(j + 1, 0)

        if nch % 2 == 0:
            body(nch - 1, 1)
        plsc.subcore_barrier()
        pltpu.sync_copy(
            acc.at[pl.ds(sid * rpt, rpt)],
            out_hbm.at[pl.ds(cid * n + sid * rpt, rpt)],
        )
        if tail:
            @pl.when(sid == 0)
            def _out_tail():
                pltpu.sync_copy(
                    acc.at[pl.ds(_NS * rpt, tail)],
                    out_hbm.at[pl.ds(cid * n + _NS * rpt, tail)],
                )

    return scatter


def kernel(x, edge_index, edge_attr, W1, b1, W2, b2, Wih, Whh, bih, bhh):
    n, h = x.shape
    e = edge_index.shape[1]
    src = edge_index[0].astype(jnp.int32)
    dst = edge_index[1].astype(jnp.int32)

    wa = W1[:, :h].T
    wb = W1[:, h:2 * h].T
    wc = W1[:, 2 * h:].T
    w2t = W2.T
    whht = Whh.T
    wiht = Wih.T
    wcat = jnp.concatenate([whht, wa, wb], axis=1)
    bcat = jnp.concatenate(
        [bhh, jnp.zeros((2 * h,), jnp.float32)]
    ).reshape(1, 5 * h)

    gh, xa, xb = _node_linear(x, wcat, bcat, block_rows=2000)

    wchunk = _NW * _CHUNK
    tw = e // wchunk
    cuts = [tw * i // _NSLICE for i in range(_NSLICE + 1)]
    zeros = jnp.zeros((n, h), jnp.float32)
    parts = []
    for i in range(_NSLICE):
        lo, es = cuts[i] * wchunk, (cuts[i + 1] - cuts[i]) * wchunk
        sl = slice(lo, lo + es)
        g_i = _make_gather(n, es, h)(xa, xb, src[sl], dst[sl])
        msg_i = _edge_mlp(
            g_i, edge_attr[sl], wc, b1.reshape(1, h), w2t, b2.reshape(1, h),
            block_rows=wchunk,
        )
        dst3d_i = dst[sl].reshape(_NW, es // wchunk, _CHUNK)
        parts.append(_make_scatter(n, es, h)(msg_i, dst3d_i, zeros))

    return _gru(parts, gh, x, wiht, bih.reshape(1, 3 * h), block_rows=2000)


# speedup vs baseline: 1.1983x; 1.0704x over previous
import functools

import jax
import jax.numpy as jnp
from jax import lax
from jax.experimental import pallas as pl
from jax.experimental.pallas import tpu as pltpu
from jax.experimental.pallas import tpu_sc as plsc

_NC = 2
_NS = 16
_NW = _NC * _NS
_CHUNK = 80
_NSLICE = 1


def _linear_body(x_ref, w_ref, b_ref, gh_ref, xa_ref, xb_ref):
    h = x_ref.shape[1]
    out = (
        jnp.dot(x_ref[...], w_ref[...], preferred_element_type=jnp.float32)
        + b_ref[...]
    )
    gh_ref[...] = out[:, :3 * h]
    xa_ref[...] = out[:, 3 * h:4 * h]
    xb_ref[...] = out[:, 4 * h:]


def _node_linear(x, wcat, bcat, block_rows, interpret=False):
    n, h = x.shape
    ho = wcat.shape[1]
    row = lambda w: pl.BlockSpec((block_rows, w), lambda i: (i, 0))
    return pl.pallas_call(
        _linear_body,
        grid=(n // block_rows,),
        in_specs=[
            row(h),
            pl.BlockSpec((h, ho), lambda i: (0, 0)),
            pl.BlockSpec((1, ho), lambda i: (0, 0)),
        ],
        out_specs=[row(3 * h), row(h), row(h)],
        out_shape=[
            jax.ShapeDtypeStruct((n, 3 * h), jnp.float32),
            jax.ShapeDtypeStruct((n, h), jnp.float32),
            jax.ShapeDtypeStruct((n, h), jnp.float32),
        ],
        interpret=interpret,
    )(x, wcat, bcat)


def _edge_mlp_body(g_ref, ea_ref, wc_ref, b1_ref, w2_ref, b2_ref, o_ref):
    pre = (
        g_ref[...]
        + jnp.dot(ea_ref[...], wc_ref[...], preferred_element_type=jnp.float32)
        + b1_ref[...]
    )
    hdn = jnp.maximum(pre, 0.0)
    o_ref[...] = (
        jnp.dot(hdn, w2_ref[...], preferred_element_type=jnp.float32) + b2_ref[...]
    )


def _edge_mlp(g, ea, wc, b1, w2, b2, block_rows, interpret=False):
    e, h = ea.shape
    row_spec = pl.BlockSpec((block_rows, h), lambda i: (i, 0))
    full = lambda shape: pl.BlockSpec(shape, lambda i: (0, 0))
    return pl.pallas_call(
        _edge_mlp_body,
        grid=(e // block_rows,),
        in_specs=[
            row_spec, row_spec,
            full((h, h)), full((1, h)), full((h, h)), full((1, h)),
        ],
        out_specs=row_spec,
        out_shape=jax.ShapeDtypeStruct((e, h), jnp.float32),
        interpret=interpret,
    )(g, ea, wc, b1, w2, b2)


def _gru(parts, gh, x, wih, bih, block_rows, interpret=False):
    n, h = x.shape
    nb = n // block_rows
    npart = len(parts)

    def body(*refs):
        part_refs = refs[:2 * npart]
        gh_ref, x_ref, wih_ref, bih_ref, o_ref = refs[2 * npart:]
        agg = part_refs[0][...]
        for p in part_refs[1:]:
            agg = agg + p[...]
        gi = (
            jnp.dot(agg, wih_ref[...], preferred_element_type=jnp.float32)
            + bih_ref[...]
        )
        ghv = gh_ref[...]
        r = jax.nn.sigmoid(gi[:, :h] + ghv[:, :h])
        z = jax.nn.sigmoid(gi[:, h:2 * h] + ghv[:, h:2 * h])
        nn = jnp.tanh(gi[:, 2 * h:] + r * ghv[:, 2 * h:])
        o_ref[...] = (1.0 - z) * nn + z * x_ref[...]

    row_spec = pl.BlockSpec((block_rows, h), lambda i: (i, 0))
    row3_spec = pl.BlockSpec((block_rows, 3 * h), lambda i: (i, 0))
    lo_spec = pl.BlockSpec((block_rows, h), lambda i: (i + nb, 0))
    full = lambda shape: pl.BlockSpec(shape, lambda i: (0, 0))
    part_specs, part_args = [], []
    for p in parts:
        part_specs += [row_spec, lo_spec]
        part_args += [p, p]
    return pl.pallas_call(
        body,
        grid=(nb,),
        in_specs=part_specs + [
            row3_spec, row_spec, full((h, 3 * h)), full((1, 3 * h)),
        ],
        out_specs=row_spec,
        out_shape=jax.ShapeDtypeStruct((n, h), jnp.float32),
        interpret=interpret,
    )(*part_args, gh, x, wih, bih)


def _make_gather(n, e, h):
    epw = e // _NW
    nch = epw // _CHUNK
    assert epw % _CHUNK == 0
    mesh = plsc.VectorSubcoreMesh(core_axis_name="c", subcore_axis_name="s")

    @functools.partial(
        pl.kernel,
        out_type=jax.ShapeDtypeStruct((e, h), jnp.float32),
        mesh=mesh,
        scratch_types=[
            pltpu.VMEM((epw,), jnp.int32),
            pltpu.VMEM((epw,), jnp.int32),
            pltpu.VMEM((2, _CHUNK, h), jnp.float32),
            pltpu.VMEM((2, _CHUNK, h), jnp.float32),
            pltpu.SemaphoreType.DMA,
            pltpu.SemaphoreType.DMA,
            pltpu.SemaphoreType.DMA,
            pltpu.SemaphoreType.DMA,
        ],
    )
    def gather(xa_hbm, xb_hbm, src_hbm, dst_hbm, g_hbm,
               sidx, didx, srows, drows, gsem0, gsem1, wsem0, wsem1):
        wid = lax.axis_index("s") * _NC + lax.axis_index("c")
        base0 = wid * epw
        pltpu.sync_copy(src_hbm.at[pl.ds(base0, epw)], sidx)
        pltpu.sync_copy(dst_hbm.at[pl.ds(base0, epw)], didx)
        gsems = (gsem0, gsem1)
        wsems = (wsem0, wsem1)

        def start_gather(j, b):
            off = j * _CHUNK
            pltpu.async_copy(
                xa_hbm.at[sidx.at[pl.ds(off, _CHUNK)]], srows.at[b], gsems[b])
            pltpu.async_copy(
                xb_hbm.at[didx.at[pl.ds(off, _CHUNK)]], drows.at[b], gsems[b])

        def wait_gather(b):
            pltpu.make_async_copy(
                xa_hbm.at[sidx.at[pl.ds(0, _CHUNK)]], srows.at[b],
                gsems[b]).wait()
            pltpu.make_async_copy(
                xa_hbm.at[sidx.at[pl.ds(0, _CHUNK)]], drows.at[b],
                gsems[b]).wait()

        def wait_wb(b):
            pltpu.make_async_copy(
                srows.at[b], g_hbm.at[pl.ds(base0, _CHUNK)], wsems[b]).wait()

        def body(j, b, first=False):
            @pl.when(j + 1 < nch)
            def _prefetch():
                if not first:
                    wait_wb(1 - b)
                start_gather(j + 1, 1 - b)
            wait_gather(b)

            @pl.loop(0, _CHUNK, step=8)
            def _row(r):
                for u in range(8):
                    for c in range(h // 16):
                        sl = pl.ds(c * 16, 16)
                        srows[b, r + u, sl] += drows[b, r + u, sl]

            pltpu.async_copy(
                srows.at[b], g_hbm.at[pl.ds(base0 + j * _CHUNK, _CHUNK)],
                wsems[b])

        start_gather(0, 0)
        body(0, 0, first=True)

        @pl.loop(1, 2 * ((nch - 1) // 2), step=2)
        def _pair(j):
            body(j, 1)
            body(j + 1, 0)

        if nch % 2 == 0:
            body(nch - 1, 1)
        wait_wb(0)
        wait_wb(1)

    return gather


def _make_scatter(n, e, h):
    epw = e // _NW
    nch = epw // _CHUNK
    assert epw % _CHUNK == 0
    rpt = (n // _NS) & ~7
    tail = n - _NS * rpt
    mesh = plsc.VectorSubcoreMesh(core_axis_name="c", subcore_axis_name="s")

    @functools.partial(
        pl.kernel,
        out_type=jax.ShapeDtypeStruct((_NC * n, h), jnp.float32),
        mesh=mesh,
        scratch_types=[
            pltpu.VMEM((nch, _CHUNK), jnp.int32),
            pltpu.VMEM((2, _CHUNK, h), jnp.float32),
            pltpu.VMEM_SHARED((n, h), jnp.float32),
            pltpu.SemaphoreType.DMA,
            pltpu.SemaphoreType.DMA,
        ],
    )
    def scatter(msg_hbm, dst3d_hbm, zero_hbm, out_hbm, idx, rows, acc,
                lsem0, lsem1):
        cid = lax.axis_index("c")
        sid = lax.axis_index("s")
        wid = sid * _NC + cid
        base0 = wid * epw
        lsems = (lsem0, lsem1)
        pltpu.sync_copy(dst3d_hbm.at[wid], idx)
        pltpu.sync_copy(
            zero_hbm.at[pl.ds(sid * rpt, rpt)],
            acc.at[pl.ds(sid * rpt, rpt)],
        )
        if tail:
            @pl.when(sid == 0)
            def _zero_tail():
                pltpu.sync_copy(
                    zero_hbm.at[pl.ds(_NS * rpt, tail)],
                    acc.at[pl.ds(_NS * rpt, tail)],
                )
        plsc.subcore_barrier()

        def start_load(j, b):
            pltpu.async_copy(
                msg_hbm.at[pl.ds(base0 + j * _CHUNK, _CHUNK)], rows.at[b],
                lsems[b])

        def wait_load(b):
            pltpu.make_async_copy(
                msg_hbm.at[pl.ds(base0, _CHUNK)], rows.at[b], lsems[b]).wait()

        def body(j, b):
            @pl.when(j + 1 < nch)
            def _prefetch():
                start_load(j + 1, 1 - b)
            wait_load(b)
            pltpu.sync_copy(rows.at[b], acc.at[idx.at[j]], add=True)

        start_load(0, 0)
        body(0, 0)

        @pl.loop(1, 2 * ((nch - 1) // 2), step=2)
        def _pair(j):
            body(j, 1)
            body(j + 1, 0)

        if nch % 2 == 0:
            body(nch - 1, 1)
        plsc.subcore_barrier()
        pltpu.sync_copy(
            acc.at[pl.ds(sid * rpt, rpt)],
            out_hbm.at[pl.ds(cid * n + sid * rpt, rpt)],
        )
        if tail:
            @pl.when(sid == 0)
            def _out_tail():
                pltpu.sync_copy(
                    acc.at[pl.ds(_NS * rpt, tail)],
                    out_hbm.at[pl.ds(cid * n + _NS * rpt, tail)],
                )

    return scatter


def kernel(x, edge_index, edge_attr, W1, b1, W2, b2, Wih, Whh, bih, bhh):
    n, h = x.shape
    e = edge_index.shape[1]
    src = edge_index[0].astype(jnp.int32)
    dst = edge_index[1].astype(jnp.int32)

    wa = W1[:, :h].T
    wb = W1[:, h:2 * h].T
    wc = W1[:, 2 * h:].T
    w2t = W2.T
    whht = Whh.T
    wiht = Wih.T
    wcat = jnp.concatenate([whht, wa, wb], axis=1)
    bcat = jnp.concatenate(
        [bhh, jnp.zeros((2 * h,), jnp.float32)]
    ).reshape(1, 5 * h)

    gh, xa, xb = _node_linear(x, wcat, bcat, block_rows=2000)

    wchunk = _NW * _CHUNK
    tw = e // wchunk
    cuts = [tw * i // _NSLICE for i in range(_NSLICE + 1)]
    zeros = jnp.zeros((n, h), jnp.float32)
    parts = []
    for i in range(_NSLICE):
        lo, es = cuts[i] * wchunk, (cuts[i + 1] - cuts[i]) * wchunk
        sl = slice(lo, lo + es)
        g_i = _make_gather(n, es, h)(xa, xb, src[sl], dst[sl])
        mlp_rows = 6400 if es % 6400 == 0 else wchunk
        msg_i = _edge_mlp(
            g_i, edge_attr[sl], wc, b1.reshape(1, h), w2t, b2.reshape(1, h),
            block_rows=mlp_rows,
        )
        dst3d_i = dst[sl].reshape(_NW, es // wchunk, _CHUNK)
        parts.append(_make_scatter(n, es, h)(msg_i, dst3d_i, zeros))

    return _gru(parts, gh, x, wiht, bih.reshape(1, 3 * h), block_rows=2000)


# speedup vs baseline: 1.2041x; 1.0049x over previous
import functools

import jax
import jax.numpy as jnp
from jax import lax
from jax.experimental import pallas as pl
from jax.experimental.pallas import tpu as pltpu
from jax.experimental.pallas import tpu_sc as plsc

_NC = 2
_NS = 16
_NW = _NC * _NS
_CHUNK = 80
_NSLICE = 1


def _linear_body(x_ref, w_ref, b_ref, gh_ref, xa_ref, xb_ref):
    h = x_ref.shape[1]
    out = (
        jnp.dot(x_ref[...], w_ref[...], preferred_element_type=jnp.float32)
        + b_ref[...]
    )
    gh_ref[...] = out[:, :3 * h]
    xa_ref[...] = out[:, 3 * h:4 * h]
    xb_ref[...] = out[:, 4 * h:]


def _node_linear(x, wcat, bcat, block_rows, interpret=False):
    n, h = x.shape
    ho = wcat.shape[1]
    row = lambda w: pl.BlockSpec((block_rows, w), lambda i: (i, 0))
    return pl.pallas_call(
        _linear_body,
        grid=(n // block_rows,),
        in_specs=[
            row(h),
            pl.BlockSpec((h, ho), lambda i: (0, 0)),
            pl.BlockSpec((1, ho), lambda i: (0, 0)),
        ],
        out_specs=[row(3 * h), row(h), row(h)],
        out_shape=[
            jax.ShapeDtypeStruct((n, 3 * h), jnp.float32),
            jax.ShapeDtypeStruct((n, h), jnp.float32),
            jax.ShapeDtypeStruct((n, h), jnp.float32),
        ],
        interpret=interpret,
    )(x, wcat, bcat)


def _edge_mlp_body(g_ref, ea_ref, wc_ref, b1_ref, w2_ref, b2_ref, o_ref):
    pre = (
        g_ref[...]
        + jnp.dot(ea_ref[...], wc_ref[...], preferred_element_type=jnp.float32)
        + b1_ref[...]
    )
    hdn = jnp.maximum(pre, 0.0)
    o_ref[...] = (
        jnp.dot(hdn, w2_ref[...], preferred_element_type=jnp.float32) + b2_ref[...]
    )


def _edge_mlp(g, ea, wc, b1, w2, b2, block_rows, interpret=False):
    e, h = ea.shape
    row_spec = pl.BlockSpec((block_rows, h), lambda i: (i, 0))
    full = lambda shape: pl.BlockSpec(shape, lambda i: (0, 0))
    return pl.pallas_call(
        _edge_mlp_body,
        grid=(e // block_rows,),
        in_specs=[
            row_spec, row_spec,
            full((h, h)), full((1, h)), full((h, h)), full((1, h)),
        ],
        out_specs=row_spec,
        out_shape=jax.ShapeDtypeStruct((e, h), jnp.float32),
        interpret=interpret,
    )(g, ea, wc, b1, w2, b2)


def _gru(parts, gh, x, wih, bih, block_rows, interpret=False):
    n, h = x.shape
    nb = n // block_rows
    npart = len(parts)

    def body(*refs):
        part_refs = refs[:2 * npart]
        gh_ref, x_ref, wih_ref, bih_ref, o_ref = refs[2 * npart:]
        agg = part_refs[0][...]
        for p in part_refs[1:]:
            agg = agg + p[...]
        gi = (
            jnp.dot(agg, wih_ref[...], preferred_element_type=jnp.float32)
            + bih_ref[...]
        )
        ghv = gh_ref[...]
        r = jax.nn.sigmoid(gi[:, :h] + ghv[:, :h])
        z = jax.nn.sigmoid(gi[:, h:2 * h] + ghv[:, h:2 * h])
        nn = jnp.tanh(gi[:, 2 * h:] + r * ghv[:, 2 * h:])
        o_ref[...] = (1.0 - z) * nn + z * x_ref[...]

    row_spec = pl.BlockSpec((block_rows, h), lambda i: (i, 0))
    row3_spec = pl.BlockSpec((block_rows, 3 * h), lambda i: (i, 0))
    lo_spec = pl.BlockSpec((block_rows, h), lambda i: (i + nb, 0))
    full = lambda shape: pl.BlockSpec(shape, lambda i: (0, 0))
    part_specs, part_args = [], []
    for p in parts:
        part_specs += [row_spec, lo_spec]
        part_args += [p, p]
    return pl.pallas_call(
        body,
        grid=(nb,),
        in_specs=part_specs + [
            row3_spec, row_spec, full((h, 3 * h)), full((1, 3 * h)),
        ],
        out_specs=row_spec,
        out_shape=jax.ShapeDtypeStruct((n, h), jnp.float32),
        interpret=interpret,
    )(*part_args, gh, x, wih, bih)


def _make_gather(n, e, h):
    epw = e // _NW
    nch = epw // _CHUNK
    assert epw % _CHUNK == 0
    mesh = plsc.VectorSubcoreMesh(core_axis_name="c", subcore_axis_name="s")

    @functools.partial(
        pl.kernel,
        out_type=jax.ShapeDtypeStruct((e, h), jnp.float32),
        mesh=mesh,
        scratch_types=[
            pltpu.VMEM((epw,), jnp.int32),
            pltpu.VMEM((epw,), jnp.int32),
            pltpu.VMEM((2, _CHUNK, h), jnp.float32),
            pltpu.VMEM((2, _CHUNK, h), jnp.float32),
            pltpu.SemaphoreType.DMA,
            pltpu.SemaphoreType.DMA,
            pltpu.SemaphoreType.DMA,
            pltpu.SemaphoreType.DMA,
        ],
    )
    def gather(xa_hbm, xb_hbm, src_hbm, dst_hbm, g_hbm,
               sidx, didx, srows, drows, gsem0, gsem1, wsem0, wsem1):
        wid = lax.axis_index("s") * _NC + lax.axis_index("c")
        base0 = wid * epw
        pltpu.sync_copy(src_hbm.at[pl.ds(base0, epw)], sidx)
        pltpu.sync_copy(dst_hbm.at[pl.ds(base0, epw)], didx)
        gsems = (gsem0, gsem1)
        wsems = (wsem0, wsem1)

        def start_gather(j, b):
            off = j * _CHUNK
            pltpu.async_copy(
                xa_hbm.at[sidx.at[pl.ds(off, _CHUNK)]], srows.at[b], gsems[b])
            pltpu.async_copy(
                xb_hbm.at[didx.at[pl.ds(off, _CHUNK)]], drows.at[b], gsems[b])

        def wait_gather(b):
            pltpu.make_async_copy(
                xa_hbm.at[sidx.at[pl.ds(0, _CHUNK)]], srows.at[b],
                gsems[b]).wait()
            pltpu.make_async_copy(
                xa_hbm.at[sidx.at[pl.ds(0, _CHUNK)]], drows.at[b],
                gsems[b]).wait()

        def wait_wb(b):
            pltpu.make_async_copy(
                srows.at[b], g_hbm.at[pl.ds(base0, _CHUNK)], wsems[b]).wait()

        def body(j, b, first=False):
            @pl.when(j + 1 < nch)
            def _prefetch():
                if not first:
                    wait_wb(1 - b)
                start_gather(j + 1, 1 - b)
            wait_gather(b)

            @pl.loop(0, _CHUNK, step=8)
            def _row(r):
                for u in range(8):
                    for c in range(h // 16):
                        sl = pl.ds(c * 16, 16)
                        srows[b, r + u, sl] += drows[b, r + u, sl]

            pltpu.async_copy(
                srows.at[b], g_hbm.at[pl.ds(base0 + j * _CHUNK, _CHUNK)],
                wsems[b])

        start_gather(0, 0)
        body(0, 0, first=True)

        @pl.loop(1, 2 * ((nch - 1) // 2), step=2)
        def _pair(j):
            body(j, 1)
            body(j + 1, 0)

        if nch % 2 == 0:
            body(nch - 1, 1)
        wait_wb(0)
        wait_wb(1)

    return gather


def _make_scatter(n, e, h):
    epw = e // _NW
    nch = epw // _CHUNK
    assert epw % _CHUNK == 0
    rpt = (n // _NS) & ~7
    tail = n - _NS * rpt
    mesh = plsc.VectorSubcoreMesh(core_axis_name="c", subcore_axis_name="s")

    @functools.partial(
        pl.kernel,
        out_type=jax.ShapeDtypeStruct((_NC * n, h), jnp.float32),
        mesh=mesh,
        scratch_types=[
            pltpu.VMEM((nch, _CHUNK), jnp.int32),
            pltpu.VMEM((2, _CHUNK, h), jnp.float32),
            pltpu.VMEM_SHARED((n, h), jnp.float32),
            pltpu.SemaphoreType.DMA,
            pltpu.SemaphoreType.DMA,
        ],
    )
    def scatter(msg_hbm, dst3d_hbm, zero_hbm, out_hbm, idx, rows, acc,
                lsem0, lsem1):
        cid = lax.axis_index("c")
        sid = lax.axis_index("s")
        wid = sid * _NC + cid
        base0 = wid * epw
        lsems = (lsem0, lsem1)
        pltpu.sync_copy(dst3d_hbm.at[wid], idx)
        pltpu.sync_copy(
            zero_hbm.at[pl.ds(sid * rpt, rpt)],
            acc.at[pl.ds(sid * rpt, rpt)],
        )
        if tail:
            @pl.when(sid == 0)
            def _zero_tail():
                pltpu.sync_copy(
                    zero_hbm.at[pl.ds(_NS * rpt, tail)],
                    acc.at[pl.ds(_NS * rpt, tail)],
                )
        plsc.subcore_barrier()

        def start_load(j, b):
            pltpu.async_copy(
                msg_hbm.at[pl.ds(base0 + j * _CHUNK, _CHUNK)], rows.at[b],
                lsems[b])

        def wait_load(b):
            pltpu.make_async_copy(
                msg_hbm.at[pl.ds(base0, _CHUNK)], rows.at[b], lsems[b]).wait()

        def body(j, b):
            @pl.when(j + 1 < nch)
            def _prefetch():
                start_load(j + 1, 1 - b)
            wait_load(b)
            pltpu.sync_copy(rows.at[b], acc.at[idx.at[j]], add=True)

        start_load(0, 0)
        body(0, 0)

        @pl.loop(1, 2 * ((nch - 1) // 2), step=2)
        def _pair(j):
            body(j, 1)
            body(j + 1, 0)

        if nch % 2 == 0:
            body(nch - 1, 1)
        plsc.subcore_barrier()
        pltpu.sync_copy(
            acc.at[pl.ds(sid * rpt, rpt)],
            out_hbm.at[pl.ds(cid * n + sid * rpt, rpt)],
        )
        if tail:
            @pl.when(sid == 0)
            def _out_tail():
                pltpu.sync_copy(
                    acc.at[pl.ds(_NS * rpt, tail)],
                    out_hbm.at[pl.ds(cid * n + _NS * rpt, tail)],
                )

    return scatter


def kernel(x, edge_index, edge_attr, W1, b1, W2, b2, Wih, Whh, bih, bhh):
    n, h = x.shape
    e = edge_index.shape[1]
    src = edge_index[0].astype(jnp.int32)
    dst = edge_index[1].astype(jnp.int32)

    wa = W1[:, :h].T
    wb = W1[:, h:2 * h].T
    wc = W1[:, 2 * h:].T
    w2t = W2.T
    whht = Whh.T
    wiht = Wih.T
    wcat = jnp.concatenate([whht, wa, wb], axis=1)
    bcat = jnp.concatenate(
        [bhh, jnp.zeros((2 * h,), jnp.float32)]
    ).reshape(1, 5 * h)

    gh, xa, xb = _node_linear(x, wcat, bcat, block_rows=2000)

    wchunk = _NW * _CHUNK
    tw = e // wchunk
    cuts = [tw * i // _NSLICE for i in range(_NSLICE + 1)]
    zeros = jnp.zeros((n, h), jnp.float32)
    parts = []
    for i in range(_NSLICE):
        lo, es = cuts[i] * wchunk, (cuts[i + 1] - cuts[i]) * wchunk
        sl = slice(lo, lo + es)
        g_i = _make_gather(n, es, h)(xa, xb, src[sl], dst[sl])
        mlp_rows = 12800 if es % 12800 == 0 else wchunk
        msg_i = _edge_mlp(
            g_i, edge_attr[sl], wc, b1.reshape(1, h), w2t, b2.reshape(1, h),
            block_rows=mlp_rows,
        )
        dst3d_i = dst[sl].reshape(_NW, es // wchunk, _CHUNK)
        parts.append(_make_scatter(n, es, h)(msg_i, dst3d_i, zeros))

    return _gru(parts, gh, x, wiht, bih.reshape(1, 3 * h), block_rows=2000)
